# Initial kernel scaffold; baseline (speedup 1.0000x reference)
import jax
import jax.numpy as jnp
from jax.experimental import pallas as pl


def kernel(logits, temperature, top_k, top_p, gumbel_u, num_logprobs):
    raise NotImplementedError("write your pallas kernel here")


# speedup vs baseline: 23.9119x; 23.9119x over previous
import jax
import jax.numpy as jnp
from jax.experimental import pallas as pl

_EPS = 1e-5
_RMAX = 0.9999999403953552
_RMAX_LOG = -5.960464477539063e-08
_NLP = 20
_BR = 8


def _key_from_f32(x):
    b = jax.lax.bitcast_convert_type(x, jnp.int32)
    return jnp.where(b >= 0, b, b ^ jnp.int32(0x7FFFFFFF))


def _f32_from_key(k):
    b = jnp.where(k >= 0, k, k ^ jnp.int32(0x7FFFFFFF))
    return jax.lax.bitcast_convert_type(b, jnp.float32)


def _avg_keys(lo, hi):
    return (lo >> 1) + (hi >> 1) + (lo & hi & 1)


def _sampler_kernel(logits_ref, u_ref, temp_ref, topk_ref, topp_ref,
                    ids_ref, idx_ref, lp_ref, rank_ref):
    x = logits_ref[...]
    temp = temp_ref[...]
    kk = topk_ref[...]
    topp = topp_ref[...]

    teff = jnp.where(temp < _EPS, 1.0, temp)
    x = x / teff

    vp = x.shape[1]
    iota = jax.lax.broadcasted_iota(jnp.int32, x.shape, 1)

    rmax = jnp.max(x, axis=1, keepdims=True)
    rmin = jnp.min(x, axis=1, keepdims=True)

    lo0 = _key_from_f32(rmin)
    hi0 = _key_from_f32(rmax) + 1

    def bs_topk(_, carry):
        lo, hi = carry
        mid = _avg_keys(lo, hi)
        t = _f32_from_key(mid)
        c = jnp.sum((x >= t).astype(jnp.int32), axis=1, keepdims=True)
        ge = c >= kk
        return jnp.where(ge, mid, lo), jnp.where(ge, hi, mid)

    lo, _ = jax.lax.fori_loop(0, 32, bs_topk, (lo0, hi0))
    thr = _f32_from_key(lo)
    y = jnp.where(x < thr, -jnp.inf, x)

    e1 = jnp.exp(y - rmax)
    s1 = jnp.sum(e1, axis=1, keepdims=True)
    p1 = e1 / s1

    ninf = jnp.full_like(rmin, -jnp.inf)
    lo0p = _key_from_f32(ninf)
    hi0p = _key_from_f32(rmax) + 1

    def bs_topp(_, carry):
        lo_, hi_ = carry
        mid = _avg_keys(lo_, hi_)
        t = _f32_from_key(mid)
        g = jnp.sum(jnp.where(y > t, p1, 0.0), axis=1, keepdims=True)
        ge = g >= topp
        return jnp.where(ge, mid, lo_), jnp.where(ge, hi_, mid)

    _, hi_p = jax.lax.fori_loop(0, 32, bs_topp, (lo0p, hi0p))
    cp = _f32_from_key(hi_p)

    gcp = jnp.sum(jnp.where(y > cp, p1, 0.0), axis=1, keepdims=True)
    lo_i0 = jnp.full_like(hi_p, -1)
    hi_i0 = jnp.full_like(hi_p, vp)

    def bs_idx(_, carry):
        lo_, hi_ = carry
        mid = (lo_ + hi_) // 2
        mass_above = jnp.sum(
            jnp.where((y == cp) & (iota > mid), p1, 0.0),
            axis=1, keepdims=True)
        pred = (gcp + mass_above) < topp
        return jnp.where(pred, lo_, mid), jnp.where(pred, mid, hi_)

    _, ib = jax.lax.fori_loop(0, 17, bs_idx, (lo_i0, hi_i0))
    keep = (y > cp) | ((y == cp) & (iota >= ib))
    x2 = jnp.where(keep, y, -jnp.inf)

    e2 = jnp.exp(x2 - rmax)
    s2 = jnp.sum(e2, axis=1, keepdims=True)
    probs = e2 / s2
    logp = (x2 - rmax) - jnp.log(s2)

    u = u_ref[...]
    q = -jnp.where(u >= _RMAX, _RMAX_LOG, jnp.log(u))
    pert = jnp.where(temp < _EPS, probs, probs / q)
    mp = jnp.max(pert, axis=1, keepdims=True)
    samp = jnp.min(jnp.where(pert == mp, iota, vp), axis=1, keepdims=True)

    slp = jnp.max(jnp.where(iota == samp, logp, -jnp.inf), axis=1, keepdims=True)
    rank = jnp.sum((logp > slp).astype(jnp.int32), axis=1, keepdims=True)

    lane = jax.lax.broadcasted_iota(jnp.int32, (x.shape[0], 128), 1)
    idx0 = jnp.where(lane == 0, samp, 0)
    lp0 = jnp.where(lane == 0, slp, 0.0)
    removed0 = jnp.zeros(x.shape, dtype=jnp.int32)

    def topn_body(j, carry):
        removed, idxa, lpa = carry
        vals = jnp.where(removed > 0, -jnp.inf, logp)
        m = jnp.max(vals, axis=1, keepdims=True)
        ji = jnp.min(
            jnp.where((vals == m) & (removed == 0), iota, vp),
            axis=1, keepdims=True)
        removed = jnp.where(iota == ji, 1, removed)
        idxa = jnp.where(lane == (j + 1), ji, idxa)
        lpa = jnp.where(lane == (j + 1), m, lpa)
        return removed, idxa, lpa

    _, idxa, lpa = jax.lax.fori_loop(0, _NLP, topn_body, (removed0, idx0, lp0))

    ids_ref[...] = jnp.where(lane == 0, samp, 0)
    idx_ref[...] = idxa
    lp_ref[...] = lpa
    rank_ref[...] = jnp.where(lane == 0, rank, 0)


def kernel(logits, temperature, top_k, top_p, gumbel_u, num_logprobs):
    B, V = logits.shape
    vp = ((V + 127) // 128) * 128
    pad = vp - V
    xl = jnp.pad(logits.astype(jnp.float32), ((0, 0), (0, pad)),
                 constant_values=-jnp.inf)
    xu = jnp.pad(gumbel_u, ((0, 0), (0, pad)), constant_values=0.5)
    t = temperature.reshape(B, 1)
    k = top_k.reshape(B, 1)
    p = top_p.reshape(B, 1)

    grid = (B // _BR,)
    ids, idxs, lps, ranks = pl.pallas_call(
        _sampler_kernel,
        grid=grid,
        in_specs=[
            pl.BlockSpec((_BR, vp), lambda i: (i, 0)),
            pl.BlockSpec((_BR, vp), lambda i: (i, 0)),
            pl.BlockSpec((_BR, 1), lambda i: (i, 0)),
            pl.BlockSpec((_BR, 1), lambda i: (i, 0)),
            pl.BlockSpec((_BR, 1), lambda i: (i, 0)),
        ],
        out_specs=[
            pl.BlockSpec((_BR, 128), lambda i: (i, 0)),
            pl.BlockSpec((_BR, 128), lambda i: (i, 0)),
            pl.BlockSpec((_BR, 128), lambda i: (i, 0)),
            pl.BlockSpec((_BR, 128), lambda i: (i, 0)),
        ],
        out_shape=[
            jax.ShapeDtypeStruct((B, 128), jnp.int32),
            jax.ShapeDtypeStruct((B, 128), jnp.int32),
            jax.ShapeDtypeStruct((B, 128), jnp.float32),
            jax.ShapeDtypeStruct((B, 128), jnp.int32),
        ],
    )(xl, xu, t, k, p)

    return ids[:, :1], idxs[:, :_NLP + 1], lps[:, :_NLP + 1], ranks[:, 0]
